# Initial kernel scaffold; baseline (speedup 1.0000x reference)
import functools

import jax
import jax.numpy as jnp
from jax import lax
from jax.experimental import pallas as pl
from jax.experimental.pallas import tpu as pltpu
from jax.experimental.pallas import tpu_sc as plsc

B = 4096
T = 50
D = 64
N = B * T
NUM_CORES = 2
NUM_SUBCORES = 16
NW = NUM_CORES * NUM_SUBCORES
PER_W = N // NW
CHUNK = 800
NCHUNK = PER_W // CHUNK

_mesh = plsc.VectorSubcoreMesh(core_axis_name="c", subcore_axis_name="s")


@functools.partial(
    pl.kernel,
    out_type=jax.ShapeDtypeStruct((N, D), jnp.float32),
    mesh=_mesh,
    scratch_types=[
        pltpu.VMEM((CHUNK,), jnp.int32),
        pltpu.VMEM((CHUNK,), jnp.float32),
        pltpu.VMEM((CHUNK, D), jnp.float32),
        pltpu.SemaphoreType.DMA,
    ],
)
def _tag_embedding(tags_hbm, probs_hbm, table_hbm, out_hbm,
                   idx_v, p_v, rows_v, sem):
    wid = lax.axis_index("s") * NUM_CORES + lax.axis_index("c")
    base = wid * PER_W

    def chunk_body(c, carry):
        off = base + c * CHUNK
        pltpu.sync_copy(tags_hbm.at[pl.ds(off, CHUNK)], idx_v)
        pltpu.sync_copy(probs_hbm.at[pl.ds(off, CHUNK)], p_v)
        pltpu.async_copy(table_hbm.at[idx_v], rows_v, sem).wait()

        def row_body(r, carry2):
            p = p_v[r]
            for j in range(D // 16):
                sl = pl.ds(j * 16, 16)
                rows_v[r, sl] = rows_v[r, sl] * p
            return carry2

        lax.fori_loop(0, CHUNK, row_body, 0, unroll=2)
        pltpu.sync_copy(rows_v, out_hbm.at[pl.ds(off, CHUNK)])
        return carry

    lax.fori_loop(0, NCHUNK, chunk_body, 0)


def kernel(tags, probs, table):
    tags_f = tags.reshape(N).astype(jnp.int32)
    probs_f = probs.reshape(N)
    out = _tag_embedding(tags_f, probs_f, table)
    return out.reshape(B, T, D)


# speedup vs baseline: 3.9993x; 3.9993x over previous
import functools

import jax
import jax.numpy as jnp
from jax import lax
from jax.experimental import pallas as pl
from jax.experimental.pallas import tpu as pltpu
from jax.experimental.pallas import tpu_sc as plsc

B = 4096
T = 50
D = 64
N = B * T
NUM_CORES = 2
NUM_SUBCORES = 16
NW = NUM_CORES * NUM_SUBCORES
PER_W = N // NW
CHUNK = 800
NCHUNK = PER_W // CHUNK

_mesh = plsc.VectorSubcoreMesh(core_axis_name="c", subcore_axis_name="s")


@functools.partial(
    pl.kernel,
    out_type=jax.ShapeDtypeStruct((N, D), jnp.float32),
    mesh=_mesh,
    scratch_types=[
        pltpu.VMEM((CHUNK,), jnp.int32),
        pltpu.VMEM((CHUNK,), jnp.float32),
        pltpu.VMEM((CHUNK, D), jnp.float32),
        pltpu.SemaphoreType.DMA,
    ],
    compiler_params=pltpu.CompilerParams(use_tc_tiling_on_sc=False),
)
def _tag_embedding(tags_hbm, probs_hbm, table_hbm, out_hbm,
                   idx_v, p_v, rows_v, sem):
    wid = lax.axis_index("s") * NUM_CORES + lax.axis_index("c")
    base = wid * PER_W

    def chunk_body(c, carry):
        off = base + c * CHUNK
        pltpu.sync_copy(tags_hbm.at[pl.ds(off, CHUNK)], idx_v)
        pltpu.sync_copy(probs_hbm.at[pl.ds(off, CHUNK)], p_v)
        pltpu.async_copy(table_hbm.at[idx_v], rows_v, sem).wait()

        def group_body(g, carry2):
            base_r = g * 16
            pv = p_v[pl.ds(base_r, 16)]
            for i in range(16):
                p = pv[i]
                r = base_r + i
                for j in range(D // 16):
                    sl = pl.ds(j * 16, 16)
                    rows_v[r, sl] = rows_v[r, sl] * p
            return carry2

        lax.fori_loop(0, CHUNK // 16, group_body, 0)
        pltpu.sync_copy(rows_v, out_hbm.at[pl.ds(off, CHUNK)])
        return carry

    lax.fori_loop(0, NCHUNK, chunk_body, 0)


def kernel(tags, probs, table):
    tags_f = tags.reshape(N).astype(jnp.int32)
    probs_f = probs.reshape(N)
    out = _tag_embedding(tags_f, probs_f, table)
    return out.reshape(B, T, D)


# speedup vs baseline: 4.5480x; 1.1372x over previous
import functools

import jax
import jax.numpy as jnp
from jax import lax
from jax.experimental import pallas as pl
from jax.experimental.pallas import tpu as pltpu
from jax.experimental.pallas import tpu_sc as plsc

B = 4096
T = 50
D = 64
N = B * T
NUM_CORES = 2
NUM_SUBCORES = 16
NW = NUM_CORES * NUM_SUBCORES
PER_W = N // NW
CHUNK = 400
NCHUNK = PER_W // CHUNK
NBUF = 3

_mesh = plsc.VectorSubcoreMesh(core_axis_name="c", subcore_axis_name="s")


@functools.partial(
    pl.kernel,
    out_type=jax.ShapeDtypeStruct((N, D), jnp.float32),
    mesh=_mesh,
    scratch_types=[
        pltpu.VMEM((NCHUNK, CHUNK), jnp.int32),
        pltpu.VMEM((NCHUNK, CHUNK), jnp.float32),
        pltpu.VMEM((NBUF, CHUNK, D), jnp.float32),
        pltpu.SemaphoreType.DMA,
        pltpu.SemaphoreType.DMA,
        pltpu.SemaphoreType.DMA,
        pltpu.SemaphoreType.DMA,
        pltpu.SemaphoreType.DMA,
        pltpu.SemaphoreType.DMA,
    ],
    compiler_params=pltpu.CompilerParams(use_tc_tiling_on_sc=False),
)
def _tag_embedding(tags_hbm, probs_hbm, table_hbm, out_hbm,
                   idx_all, p_all, rows_v, *sems):
    sem_g = sems[:NBUF]
    sem_st = sems[NBUF:]
    wid = lax.axis_index("s") * NUM_CORES + lax.axis_index("c")
    base = wid * PER_W

    pltpu.sync_copy(tags_hbm.at[pl.ds(wid * NCHUNK, NCHUNK)], idx_all)
    pltpu.sync_copy(probs_hbm.at[pl.ds(wid * NCHUNK, NCHUNK)], p_all)

    def gather_start(c):
        b = c % NBUF
        return pltpu.async_copy(table_hbm.at[idx_all.at[c]],
                                rows_v.at[b], sem_g[b])

    def store_start(c):
        b = c % NBUF
        return pltpu.async_copy(rows_v.at[b],
                                out_hbm.at[pl.ds(base + c * CHUNK, CHUNK)],
                                sem_st[b])

    def scale(c):
        b = c % NBUF

        def group_body(g, carry):
            pv = p_all[c, pl.ds(g * 16, 16)]
            for i in range(16):
                p = pv[i]
                for j in range(D // 16):
                    sl = pl.ds(j * 16, 16)
                    rows_v[b, g * 16 + i, sl] = rows_v[b, g * 16 + i, sl] * p
            return carry

        lax.fori_loop(0, CHUNK // 16, group_body, 0)

    copies_g = {}
    copies_st = {}
    copies_g[0] = gather_start(0)
    for c in range(NCHUNK):
        if c + 1 < NCHUNK:
            if c + 1 >= NBUF:
                copies_st[c + 1 - NBUF].wait()
            copies_g[c + 1] = gather_start(c + 1)
        copies_g[c].wait()
        scale(c)
        copies_st[c] = store_start(c)
    for c in range(max(0, NCHUNK - NBUF), NCHUNK):
        copies_st[c].wait()


def kernel(tags, probs, table):
    tags_f = tags.reshape(NW * NCHUNK, CHUNK).astype(jnp.int32)
    probs_f = probs.reshape(NW * NCHUNK, CHUNK)
    out = _tag_embedding(tags_f, probs_f, table)
    return out.reshape(B, T, D)
